# Initial kernel scaffold; baseline (speedup 1.0000x reference)
import math

import jax
import jax.numpy as jnp
from jax.experimental import pallas as pl
from jax.experimental.pallas import tpu as pltpu
from jax.experimental.pallas import tpu_sc as plsc
from jax.scipy.special import gammaln

D = 1536
H = 8
DK = 64
DV = 64
NRPF = 192
NE = 8
TK = 2
N = 2048
NR = 2 * N
TI = 256
TP = 256
TG = 128
NT = NE + (TK * N - NE) // TG
NP = NT * TG
SCW = 16

PREC = jax.lax.Precision.HIGHEST

_vector_mesh = plsc.VectorSubcoreMesh(core_axis_name="c", subcore_axis_name="s")


def _pos_embed(n, feature_size):
    distances = jnp.arange(-n + 1, n)
    nb = feature_size // 6
    absd = jnp.abs(distances).astype(jnp.float32)
    max_range = math.log(n) / math.log(2.0)
    half_life = 2.0 ** jnp.linspace(3.0, max_range, nb)
    f_exp = jnp.exp(-math.log(2.0) / half_life[None, :] * absd[:, None])
    cw = 2.0 ** jnp.arange(1, nb + 1).astype(jnp.float32) - 1.0
    f_cm = (cw[None, :] > absd[:, None]).astype(jnp.float32)
    stddev = n / (2.0 * nb)
    start_mean = n / float(nb)
    mean = jnp.linspace(start_mean, float(n), nb)[None, :]
    conc = (mean / stddev) ** 2
    rate = mean / (stddev ** 2)
    xpos = absd[:, None]
    log_unnorm = (conc - 1.0) * jnp.log(xpos) - rate * xpos
    log_norm = gammaln(conc) - conc * jnp.log(rate)
    probs = jnp.exp(log_unnorm - log_norm) + 1e-8
    f_g = probs / jnp.max(probs, axis=-1, keepdims=True)
    emb = jnp.concatenate([f_exp, f_cm, f_g], axis=-1)
    emb = jnp.concatenate(
        [emb, jnp.sign(distances).astype(jnp.float32)[:, None] * emb], axis=-1)
    return emb


def _qkv_body(x_ref, g_ref, b_ref, wq_ref, wk_ref, wv_ref, q_ref, k_ref, v_ref):
    xb = x_ref[...]
    m = jnp.mean(xb, axis=1, keepdims=True)
    xc = xb - m
    var = jnp.mean(xc * xc, axis=1, keepdims=True)
    xn = xc * jax.lax.rsqrt(var + 1e-5) * g_ref[...] + b_ref[...]
    q_ref[...] = jnp.dot(xn, wq_ref[...], preferred_element_type=jnp.float32,
                         precision=PREC) * (DK ** -0.5)
    k_ref[...] = jnp.dot(xn, wk_ref[...], preferred_element_type=jnp.float32,
                         precision=PREC)
    v_ref[...] = jnp.dot(xn, wv_ref[...], preferred_element_type=jnp.float32,
                         precision=PREC)


def _qkv(x2d, ln1_g, ln1_b, Wq, Wk, Wv):
    out = jax.ShapeDtypeStruct((N, H * DK), jnp.float32)
    return pl.pallas_call(
        _qkv_body,
        grid=(N // TP,),
        in_specs=[
            pl.BlockSpec((TP, D), lambda i: (i, 0)),
            pl.BlockSpec((1, D), lambda i: (0, 0)),
            pl.BlockSpec((1, D), lambda i: (0, 0)),
            pl.BlockSpec((D, H * DK), lambda i: (0, 0)),
            pl.BlockSpec((D, H * DK), lambda i: (0, 0)),
            pl.BlockSpec((D, H * DV), lambda i: (0, 0)),
        ],
        out_specs=[
            pl.BlockSpec((TP, H * DK), lambda i: (i, 0)),
            pl.BlockSpec((TP, H * DK), lambda i: (i, 0)),
            pl.BlockSpec((TP, H * DV), lambda i: (i, 0)),
        ],
        out_shape=[out, out, out],
    )(x2d, ln1_g.reshape(1, D), ln1_b.reshape(1, D), Wq, Wk, Wv)


def _relk_body(p_ref, w_ref, o_ref):
    o_ref[...] = jnp.dot(p_ref[...], w_ref[...],
                         preferred_element_type=jnp.float32, precision=PREC)


def _relk(posp, Wrel):
    return pl.pallas_call(
        _relk_body,
        out_shape=jax.ShapeDtypeStruct((NR, H * DK), jnp.float32),
    )(posp, Wrel)


def _attn_body(q_ref, k_ref, v_ref, rp_ref, rcb_ref, rpb_ref, o_ref):
    q = q_ref[...]
    qc = q + rcb_ref[...]
    qp = q + rpb_ref[...]
    kk = k_ref[...]
    rp = rp_ref[...]
    rows = []
    for bi in range(N // TI):
        qc_t = qc[bi * TI:(bi + 1) * TI, :]
        qp_t = qp[bi * TI:(bi + 1) * TI, :]
        content = jax.lax.dot_general(
            qc_t, kk, (((1,), (1,)), ((), ())),
            preferred_element_type=jnp.float32, precision=PREC)
        mf = jax.lax.dot_general(
            qp_t, rp, (((1,), (1,)), ((), ())),
            preferred_element_type=jnp.float32, precision=PREC)
        shift = (bi * TI + N) % NR
        rolled = pltpu.roll(mf, shift, 1, stride=1, stride_axis=0)
        rows.append(content + rolled[:, :N])
    logits = jnp.concatenate(rows, axis=0)
    mx = jnp.max(logits, axis=1, keepdims=True)
    el = jnp.exp(logits - mx)
    sm = jnp.sum(el, axis=1, keepdims=True)
    aw = el / sm
    o_ref[...] = jax.lax.dot_general(
        aw, v_ref[...], (((1,), (0,)), ((), ())),
        preferred_element_type=jnp.float32, precision=PREC)


def _attn(qs, k, v, Rp, rcb2, rpb2):
    return pl.pallas_call(
        _attn_body,
        grid=(H,),
        in_specs=[
            pl.BlockSpec((N, DK), lambda h: (0, h)),
            pl.BlockSpec((N, DK), lambda h: (0, h)),
            pl.BlockSpec((N, DV), lambda h: (0, h)),
            pl.BlockSpec((NR, DK), lambda h: (0, h)),
            pl.BlockSpec((1, DK), lambda h: (h, 0)),
            pl.BlockSpec((1, DK), lambda h: (h, 0)),
        ],
        out_specs=pl.BlockSpec((N, DV), lambda h: (0, h)),
        out_shape=jax.ShapeDtypeStruct((N, H * DV), jnp.float32),
    )(qs, k, v, Rp, rcb2, rpb2)


def _post_body(x_ref, a_ref, wo_ref, bo_ref, g2_ref, b2_ref, wg_ref,
               x2_ref, xn2_ref, ti_ref, gt_ref):
    x2 = x_ref[...] + jnp.dot(a_ref[...], wo_ref[...],
                              preferred_element_type=jnp.float32,
                              precision=PREC) + bo_ref[...]
    x2_ref[...] = x2
    m = jnp.mean(x2, axis=1, keepdims=True)
    xc = x2 - m
    var = jnp.mean(xc * xc, axis=1, keepdims=True)
    xn2 = xc * jax.lax.rsqrt(var + 1e-5) * g2_ref[...] + b2_ref[...]
    xn2_ref[...] = xn2
    rl = jnp.dot(xn2, wg_ref[...], preferred_element_type=jnp.float32,
                 precision=PREC)
    lane = jax.lax.broadcasted_iota(jnp.int32, rl.shape, 1)
    m1 = jnp.max(rl, axis=1, keepdims=True)
    am1 = jnp.min(jnp.where(rl == m1, lane, NE), axis=1, keepdims=True)
    rl2 = jnp.where(lane == am1, -jnp.inf, rl)
    m2 = jnp.max(rl2, axis=1, keepdims=True)
    am2 = jnp.min(jnp.where(rl2 == m2, lane, NE), axis=1, keepdims=True)
    g1 = 1.0 / (1.0 + jnp.exp(m2 - m1))
    ti_ref[...] = jnp.concatenate([am1, am2], axis=1)
    gt_ref[...] = jnp.concatenate([g1, 1.0 - g1], axis=1)


def _post(x2d, attn, Wo, bo, ln2_g, ln2_b, Wg):
    return pl.pallas_call(
        _post_body,
        grid=(N // TP,),
        in_specs=[
            pl.BlockSpec((TP, D), lambda i: (i, 0)),
            pl.BlockSpec((TP, H * DV), lambda i: (i, 0)),
            pl.BlockSpec((H * DV, D), lambda i: (0, 0)),
            pl.BlockSpec((1, D), lambda i: (0, 0)),
            pl.BlockSpec((1, D), lambda i: (0, 0)),
            pl.BlockSpec((1, D), lambda i: (0, 0)),
            pl.BlockSpec((D, NE), lambda i: (0, 0)),
        ],
        out_specs=[
            pl.BlockSpec((TP, D), lambda i: (i, 0)),
            pl.BlockSpec((TP, D), lambda i: (i, 0)),
            pl.BlockSpec((TP, TK), lambda i: (i, 0)),
            pl.BlockSpec((TP, TK), lambda i: (i, 0)),
        ],
        out_shape=[
            jax.ShapeDtypeStruct((N, D), jnp.float32),
            jax.ShapeDtypeStruct((N, D), jnp.float32),
            jax.ShapeDtypeStruct((N, TK), jnp.int32),
            jax.ShapeDtypeStruct((N, TK), jnp.float32),
        ],
    )(x2d, attn, Wo, bo.reshape(1, D), ln2_g.reshape(1, D),
      ln2_b.reshape(1, D), Wg)


def _route_body(ef_ref, p_ref, te_ref):
    ef = ef_ref[...]
    p = jnp.zeros(ef.shape, jnp.int32)
    ts_list = []
    ts = 0
    for e in range(NE):
        m = (ef == e).astype(jnp.int32)
        wr = jnp.cumsum(m, axis=1) - m
        rt = jnp.sum(m, axis=1, keepdims=True)
        ro = jnp.cumsum(rt, axis=0) - rt
        rank = wr + ro
        ne = jnp.sum(m)
        ts_list.append(ts)
        p = p + m * (rank + ts * TG)
        ts = ts + (ne + TG - 1) // TG
    p_ref[...] = p
    tt = jax.lax.broadcasted_iota(jnp.int32, (8, 128), 1)
    te = jnp.zeros((8, 128), jnp.int32)
    for e in range(1, NE):
        te = te + (tt >= ts_list[e]).astype(jnp.int32)
    te_ref[...] = te


def _route(ef):
    return pl.pallas_call(
        _route_body,
        out_shape=[
            jax.ShapeDtypeStruct((TK * N // 128, 128), jnp.int32),
            jax.ShapeDtypeStruct((8, 128), jnp.int32),
        ],
    )(ef)


def _sc_scatter(xn2, p0, p1):
    @pl.kernel(out_type=jax.ShapeDtypeStruct((NP, D), jnp.float32),
               mesh=_vector_mesh)
    def kern(x_hbm, p0_hbm, p1_hbm, o_hbm):
        def body(x_vmem, i0_vmem, i1_vmem):
            pltpu.sync_copy(x_vmem, o_hbm.at[i0_vmem.at[0]])
            pltpu.sync_copy(x_vmem, o_hbm.at[i1_vmem.at[0]])

        pltpu.emit_pipeline(
            body,
            grid=(N // SCW,),
            in_specs=[
                pl.BlockSpec((SCW, D), lambda i: (i, 0)),
                pl.BlockSpec((1, SCW), lambda i: (0, i)),
                pl.BlockSpec((1, SCW), lambda i: (0, i)),
            ],
            out_specs=[],
            core_axis_name="s",
            dimension_semantics=(pltpu.PARALLEL,),
        )(x_hbm, p0_hbm, p1_hbm)

    return kern(xn2, p0, p1)


def _sc_gather(outs, pf):
    @pl.kernel(out_type=jax.ShapeDtypeStruct((TK * N, D), jnp.float32),
               mesh=_vector_mesh)
    def kern(s_hbm, p_hbm, o_hbm):
        def body(i_vmem, o_vmem):
            pltpu.sync_copy(s_hbm.at[i_vmem.at[0]], o_vmem)

        pltpu.emit_pipeline(
            body,
            grid=(TK * N // SCW,),
            in_specs=[pl.BlockSpec((1, SCW), lambda i: (0, i))],
            out_specs=[pl.BlockSpec((SCW, D), lambda i: (i, 0))],
            core_axis_name="s",
            dimension_semantics=(pltpu.PARALLEL,),
        )(p_hbm, o_hbm)

    return kern(outs, pf)


def _gemm_body(te_ref, x_ref, w_ref, b_ref, o_ref):
    o_ref[...] = jnp.dot(x_ref[...], w_ref[0],
                         preferred_element_type=jnp.float32,
                         precision=PREC) + b_ref[...]


def _gemm(X_s, We, be, te):
    return pl.pallas_call(
        _gemm_body,
        grid_spec=pltpu.PrefetchScalarGridSpec(
            num_scalar_prefetch=1,
            grid=(NT,),
            in_specs=[
                pl.BlockSpec((TG, D), lambda t, te_r: (t, 0)),
                pl.BlockSpec((1, D, D), lambda t, te_r: (te_r[t], 0, 0)),
                pl.BlockSpec((1, D), lambda t, te_r: (te_r[t], 0)),
            ],
            out_specs=pl.BlockSpec((TG, D), lambda t, te_r: (t, 0)),
        ),
        out_shape=jax.ShapeDtypeStruct((NP, D), jnp.float32),
    )(te, X_s, We, be)


def _combine_body(x2_ref, o2_ref, gt_ref, y_ref):
    g = gt_ref[...]
    y_ref[...] = (x2_ref[...]
                  + g[:, 0:1] * o2_ref[:, 0, :]
                  + g[:, 1:2] * o2_ref[:, 1, :])


def _combine(x2, OUT2r, gt):
    return pl.pallas_call(
        _combine_body,
        grid=(N // TG,),
        in_specs=[
            pl.BlockSpec((TG, D), lambda i: (i, 0)),
            pl.BlockSpec((TG, TK, D), lambda i: (i, 0, 0)),
            pl.BlockSpec((TG, TK), lambda i: (i, 0)),
        ],
        out_specs=pl.BlockSpec((TG, D), lambda i: (i, 0)),
        out_shape=jax.ShapeDtypeStruct((N, D), jnp.float32),
    )(x2, OUT2r, gt)


def kernel(x, ln1_g, ln1_b, Wq, Wk, Wv, Wo, bo, Wrel, rcb, rpb, ln2_g, ln2_b,
           Wg, We, be):
    x2d = x.reshape(N, D)
    pos = _pos_embed(N, NRPF)
    posp = jnp.concatenate([jnp.zeros((1, NRPF), jnp.float32), pos], axis=0)

    qs, k, v = _qkv(x2d, ln1_g, ln1_b, Wq, Wk, Wv)
    Rp = _relk(posp, Wrel)
    attn = _attn(qs, k, v, Rp, rcb.reshape(H, DK), rpb.reshape(H, DK))
    x2, xn2, ti, gt = _post(x2d, attn, Wo, bo, ln2_g, ln2_b, Wg)

    ef = ti.reshape(TK * N // 128, 128)
    p, te8 = _route(ef)
    pf = p.reshape(TK * N)
    p2 = pf.reshape(N, TK)
    p0 = p2[:, 0].reshape(1, N)
    p1 = p2[:, 1].reshape(1, N)
    te = te8[0, :NT]

    X_s = _sc_scatter(xn2, p0, p1)
    OUT_s = _gemm(X_s, We, be, te)
    OUT2 = _sc_gather(OUT_s, pf.reshape(1, TK * N))
    y = _combine(x2, OUT2.reshape(N, TK, D), gt)
    return y.reshape(1, N, D)


# speedup vs baseline: 4.6185x; 4.6185x over previous
import math

import jax
import jax.numpy as jnp
from jax.experimental import pallas as pl
from jax.experimental.pallas import tpu as pltpu
from jax.experimental.pallas import tpu_sc as plsc
from jax.scipy.special import gammaln

D = 1536
H = 8
DK = 64
DV = 64
NRPF = 192
NE = 8
TK = 2
N = 2048
NR = 2 * N
TI = 256
TP = 256
TG = 128
NT = NE + (TK * N - NE) // TG
NP = NT * TG
SCW = 16

PREC = jax.lax.Precision.DEFAULT
PREC_HI = jax.lax.Precision.HIGHEST


def _vector_mesh():
    return plsc.VectorSubcoreMesh(core_axis_name="c", subcore_axis_name="s")


def _pos_embed(n, feature_size):
    distances = jnp.arange(-n + 1, n)
    nb = feature_size // 6
    absd = jnp.abs(distances).astype(jnp.float32)
    max_range = math.log(n) / math.log(2.0)
    half_life = 2.0 ** jnp.linspace(3.0, max_range, nb)
    f_exp = jnp.exp(-math.log(2.0) / half_life[None, :] * absd[:, None])
    cw = 2.0 ** jnp.arange(1, nb + 1).astype(jnp.float32) - 1.0
    f_cm = (cw[None, :] > absd[:, None]).astype(jnp.float32)
    stddev = n / (2.0 * nb)
    start_mean = n / float(nb)
    mean = jnp.linspace(start_mean, float(n), nb)[None, :]
    conc = (mean / stddev) ** 2
    rate = mean / (stddev ** 2)
    xpos = absd[:, None]
    log_unnorm = (conc - 1.0) * jnp.log(xpos) - rate * xpos
    log_norm = gammaln(conc) - conc * jnp.log(rate)
    probs = jnp.exp(log_unnorm - log_norm) + 1e-8
    f_g = probs / jnp.max(probs, axis=-1, keepdims=True)
    emb = jnp.concatenate([f_exp, f_cm, f_g], axis=-1)
    emb = jnp.concatenate(
        [emb, jnp.sign(distances).astype(jnp.float32)[:, None] * emb], axis=-1)
    return emb


def _qkv_body(x_ref, g_ref, b_ref, wq_ref, wk_ref, wv_ref, q_ref, k_ref, v_ref):
    xb = x_ref[...]
    m = jnp.mean(xb, axis=1, keepdims=True)
    xc = xb - m
    var = jnp.mean(xc * xc, axis=1, keepdims=True)
    xn = xc * jax.lax.rsqrt(var + 1e-5) * g_ref[...] + b_ref[...]
    q3 = jnp.dot(xn, wq_ref[...], preferred_element_type=jnp.float32,
                 precision=PREC) * (DK ** -0.5)
    k3 = jnp.dot(xn, wk_ref[...], preferred_element_type=jnp.float32,
                 precision=PREC)
    v3 = jnp.dot(xn, wv_ref[...], preferred_element_type=jnp.float32,
                 precision=PREC)
    for h in range(H):
        q_ref[h] = q3[:, h * DK:(h + 1) * DK]
        k_ref[h] = k3[:, h * DK:(h + 1) * DK]
        v_ref[h] = v3[:, h * DV:(h + 1) * DV]


def _qkv(x2d, ln1_g, ln1_b, Wq, Wk, Wv):
    out = jax.ShapeDtypeStruct((H, N, DK), jnp.float32)
    return pl.pallas_call(
        _qkv_body,
        grid=(N // TP,),
        in_specs=[
            pl.BlockSpec((TP, D), lambda i: (i, 0)),
            pl.BlockSpec((1, D), lambda i: (0, 0)),
            pl.BlockSpec((1, D), lambda i: (0, 0)),
            pl.BlockSpec((D, H * DK), lambda i: (0, 0)),
            pl.BlockSpec((D, H * DK), lambda i: (0, 0)),
            pl.BlockSpec((D, H * DV), lambda i: (0, 0)),
        ],
        out_specs=[
            pl.BlockSpec((H, TP, DK), lambda i: (0, i, 0)),
            pl.BlockSpec((H, TP, DK), lambda i: (0, i, 0)),
            pl.BlockSpec((H, TP, DV), lambda i: (0, i, 0)),
        ],
        out_shape=[out, out, out],
    )(x2d, ln1_g.reshape(1, D), ln1_b.reshape(1, D), Wq, Wk, Wv)


def _relk_body(p_ref, w_ref, o_ref):
    r = jnp.dot(p_ref[...], w_ref[...],
                preferred_element_type=jnp.float32, precision=PREC)
    for h in range(H):
        o_ref[h] = r[:, h * DK:(h + 1) * DK]


def _relk(posp, Wrel):
    return pl.pallas_call(
        _relk_body,
        out_shape=jax.ShapeDtypeStruct((H, NR, DK), jnp.float32),
    )(posp, Wrel)


def _attn_body(q_ref, k_ref, v_ref, rp_ref, rcb_ref, rpb_ref, o_ref):
    q = q_ref[0]
    qc = q + rcb_ref[0]
    qp = q + rpb_ref[0]
    kk = k_ref[0]
    rp = rp_ref[0]
    rows = []
    for bi in range(N // TI):
        qc_t = qc[bi * TI:(bi + 1) * TI, :]
        qp_t = qp[bi * TI:(bi + 1) * TI, :]
        content = jax.lax.dot_general(
            qc_t, kk, (((1,), (1,)), ((), ())),
            preferred_element_type=jnp.float32, precision=PREC)
        mf = jax.lax.dot_general(
            qp_t, rp, (((1,), (1,)), ((), ())),
            preferred_element_type=jnp.float32, precision=PREC)
        shift = (bi * TI + N) % NR
        rolled = pltpu.roll(mf, shift, 1, stride=1, stride_axis=0)
        rows.append(content + rolled[:, :N])
    logits = jnp.concatenate(rows, axis=0)
    mx = jnp.max(logits, axis=1, keepdims=True)
    el = jnp.exp(logits - mx)
    sm = jnp.sum(el, axis=1, keepdims=True)
    aw = el / sm
    o_ref[0] = jax.lax.dot_general(
        aw, v_ref[0], (((1,), (0,)), ((), ())),
        preferred_element_type=jnp.float32, precision=PREC)


def _attn(qs, k, v, Rp, rcb3, rpb3):
    return pl.pallas_call(
        _attn_body,
        grid=(H,),
        in_specs=[
            pl.BlockSpec((1, N, DK), lambda h: (h, 0, 0)),
            pl.BlockSpec((1, N, DK), lambda h: (h, 0, 0)),
            pl.BlockSpec((1, N, DV), lambda h: (h, 0, 0)),
            pl.BlockSpec((1, NR, DK), lambda h: (h, 0, 0)),
            pl.BlockSpec((1, 1, DK), lambda h: (h, 0, 0)),
            pl.BlockSpec((1, 1, DK), lambda h: (h, 0, 0)),
        ],
        out_specs=pl.BlockSpec((1, N, DV), lambda h: (h, 0, 0)),
        out_shape=jax.ShapeDtypeStruct((H, N, DV), jnp.float32),
    )(qs, k, v, Rp, rcb3, rpb3)


def _post_body(x_ref, a_ref, wo_ref, bo_ref, g2_ref, b2_ref, wg_ref,
               x2_ref, xn2_ref, ti_ref, gt_ref):
    x2 = x_ref[...] + jnp.dot(a_ref[...], wo_ref[...],
                              preferred_element_type=jnp.float32,
                              precision=PREC) + bo_ref[...]
    x2_ref[...] = x2
    m = jnp.mean(x2, axis=1, keepdims=True)
    xc = x2 - m
    var = jnp.mean(xc * xc, axis=1, keepdims=True)
    xn2 = xc * jax.lax.rsqrt(var + 1e-5) * g2_ref[...] + b2_ref[...]
    xn2_ref[...] = xn2
    rl = jnp.dot(xn2, wg_ref[...], preferred_element_type=jnp.float32,
                 precision=PREC)
    lane = jax.lax.broadcasted_iota(jnp.int32, rl.shape, 1)
    m1 = jnp.max(rl, axis=1, keepdims=True)
    am1 = jnp.min(jnp.where(rl == m1, lane, NE), axis=1, keepdims=True)
    rl2 = jnp.where(lane == am1, -jnp.inf, rl)
    m2 = jnp.max(rl2, axis=1, keepdims=True)
    am2 = jnp.min(jnp.where(rl2 == m2, lane, NE), axis=1, keepdims=True)
    g1 = 1.0 / (1.0 + jnp.exp(m2 - m1))
    ti_ref[...] = jnp.concatenate([am1, am2], axis=1)
    gt_ref[...] = jnp.concatenate([g1, 1.0 - g1], axis=1)


def _post(x2d, attn2, Wo, bo, ln2_g, ln2_b, Wg):
    return pl.pallas_call(
        _post_body,
        grid=(N // TP,),
        in_specs=[
            pl.BlockSpec((TP, D), lambda i: (i, 0)),
            pl.BlockSpec((TP, H * DV), lambda i: (i, 0)),
            pl.BlockSpec((H * DV, D), lambda i: (0, 0)),
            pl.BlockSpec((1, D), lambda i: (0, 0)),
            pl.BlockSpec((1, D), lambda i: (0, 0)),
            pl.BlockSpec((1, D), lambda i: (0, 0)),
            pl.BlockSpec((D, NE), lambda i: (0, 0)),
        ],
        out_specs=[
            pl.BlockSpec((TP, D), lambda i: (i, 0)),
            pl.BlockSpec((TP, D), lambda i: (i, 0)),
            pl.BlockSpec((TP, TK), lambda i: (i, 0)),
            pl.BlockSpec((TP, TK), lambda i: (i, 0)),
        ],
        out_shape=[
            jax.ShapeDtypeStruct((N, D), jnp.float32),
            jax.ShapeDtypeStruct((N, D), jnp.float32),
            jax.ShapeDtypeStruct((N, TK), jnp.int32),
            jax.ShapeDtypeStruct((N, TK), jnp.float32),
        ],
    )(x2d, attn2, Wo, bo.reshape(1, D), ln2_g.reshape(1, D),
      ln2_b.reshape(1, D), Wg)


def _route_body(ef_ref, p_ref, te_ref):
    ef = ef_ref[...]
    R, C = ef.shape
    rr = jax.lax.broadcasted_iota(jnp.int32, (C, C), 0)
    cc = jax.lax.broadcasted_iota(jnp.int32, (C, C), 1)
    Uincl = (rr <= cc).astype(jnp.float32)
    r2 = jax.lax.broadcasted_iota(jnp.int32, (R, R), 0)
    c2 = jax.lax.broadcasted_iota(jnp.int32, (R, R), 1)
    Lstrict = (c2 < r2).astype(jnp.float32)
    p = jnp.zeros(ef.shape, jnp.int32)
    ts_list = []
    ts = jnp.zeros((1, 1), jnp.int32)
    for e in range(NE):
        m = (ef == e).astype(jnp.float32)
        wr = jnp.dot(m, Uincl, preferred_element_type=jnp.float32,
                     precision=PREC_HI) - m
        rt = jnp.sum(m, axis=1, keepdims=True)
        ro = jnp.dot(Lstrict, rt, preferred_element_type=jnp.float32,
                     precision=PREC_HI)
        rank = (wr + ro).astype(jnp.int32)
        ne = jnp.sum(rt, axis=0, keepdims=True).astype(jnp.int32)
        ts_list.append(ts)
        p = p + m.astype(jnp.int32) * (rank + ts * TG)
        ts = ts + (ne + TG - 1) // TG
    p_ref[...] = p
    tt = jax.lax.broadcasted_iota(jnp.int32, (8, 128), 1)
    te = jnp.zeros((8, 128), jnp.int32)
    for e in range(1, NE):
        te = te + (tt >= ts_list[e]).astype(jnp.int32)
    te_ref[...] = te


def _route(ef):
    return pl.pallas_call(
        _route_body,
        out_shape=[
            jax.ShapeDtypeStruct((TK * N // 128, 128), jnp.int32),
            jax.ShapeDtypeStruct((8, 128), jnp.int32),
        ],
    )(ef)


_NWORK = 32


def _sc_scatter(xn2, p0, p1):
    @pl.kernel(out_type=jax.ShapeDtypeStruct((NP, D), jnp.float32),
               mesh=_vector_mesh(),
               scratch_types=[pltpu.VMEM((1, N), jnp.int32),
                              pltpu.VMEM((1, N), jnp.int32),
                              pltpu.VMEM((SCW, D), jnp.float32),
                              pltpu.SemaphoreType.DMA])
    def kern(x_hbm, p0_hbm, p1_hbm, o_hbm, i0, i1, buf, sem):
        c = jax.lax.axis_index("c")
        s = jax.lax.axis_index("s")
        w = c * 16 + s
        pltpu.async_copy(p0_hbm, i0, sem).wait()
        pltpu.async_copy(p1_hbm, i1, sem).wait()
        per = (N // SCW) // _NWORK

        @pl.loop(0, per)
        def _(j):
            t = w * per + j
            pltpu.async_copy(x_hbm.at[pl.ds(t * SCW, SCW), :], buf, sem).wait()
            pltpu.async_copy(buf, o_hbm.at[i0.at[0, pl.ds(t * SCW, SCW)]],
                             sem).wait()
            pltpu.async_copy(buf, o_hbm.at[i1.at[0, pl.ds(t * SCW, SCW)]],
                             sem).wait()

    return kern(xn2, p0, p1)


def _sc_gather(outs, pf):
    @pl.kernel(out_type=jax.ShapeDtypeStruct((TK * N, D), jnp.float32),
               mesh=_vector_mesh(),
               scratch_types=[pltpu.VMEM((1, TK * N), jnp.int32),
                              pltpu.VMEM((SCW, D), jnp.float32),
                              pltpu.SemaphoreType.DMA])
    def kern(s_hbm, p_hbm, o_hbm, idx, buf, sem):
        c = jax.lax.axis_index("c")
        s = jax.lax.axis_index("s")
        w = c * 16 + s
        pltpu.async_copy(p_hbm, idx, sem).wait()
        per = (TK * N // SCW) // _NWORK

        @pl.loop(0, per)
        def _(j):
            t = w * per + j
            pltpu.async_copy(s_hbm.at[idx.at[0, pl.ds(t * SCW, SCW)]],
                             buf, sem).wait()
            pltpu.async_copy(buf, o_hbm.at[pl.ds(t * SCW, SCW), :], sem).wait()

    return kern(outs, pf)


def _gemm_body(te_ref, x_ref, w_ref, b_ref, o_ref):
    o_ref[...] = jnp.dot(x_ref[...], w_ref[0],
                         preferred_element_type=jnp.float32,
                         precision=PREC) + b_ref[0]


def _gemm(X_s, We, be3, te):
    return pl.pallas_call(
        _gemm_body,
        grid_spec=pltpu.PrefetchScalarGridSpec(
            num_scalar_prefetch=1,
            grid=(NT,),
            in_specs=[
                pl.BlockSpec((TG, D), lambda t, te_r: (t, 0)),
                pl.BlockSpec((1, D, D), lambda t, te_r: (te_r[t], 0, 0)),
                pl.BlockSpec((1, 1, D), lambda t, te_r: (te_r[t], 0, 0)),
            ],
            out_specs=pl.BlockSpec((TG, D), lambda t, te_r: (t, 0)),
        ),
        out_shape=jax.ShapeDtypeStruct((NP, D), jnp.float32),
    )(te, X_s, We, be3)


def _combine_body(x2_ref, o2_ref, gt_ref, y_ref):
    g = gt_ref[...]
    y_ref[...] = (x2_ref[...]
                  + g[:, 0:1] * o2_ref[:, 0, :]
                  + g[:, 1:2] * o2_ref[:, 1, :])


def _combine(x2, OUT2r, gt):
    return pl.pallas_call(
        _combine_body,
        grid=(N // TG,),
        in_specs=[
            pl.BlockSpec((TG, D), lambda i: (i, 0)),
            pl.BlockSpec((TG, TK, D), lambda i: (i, 0, 0)),
            pl.BlockSpec((TG, TK), lambda i: (i, 0)),
        ],
        out_specs=pl.BlockSpec((TG, D), lambda i: (i, 0)),
        out_shape=jax.ShapeDtypeStruct((N, D), jnp.float32),
    )(x2, OUT2r, gt)


def kernel(x, ln1_g, ln1_b, Wq, Wk, Wv, Wo, bo, Wrel, rcb, rpb, ln2_g, ln2_b,
           Wg, We, be):
    x2d = x.reshape(N, D)
    pos = _pos_embed(N, NRPF)
    posp = jnp.concatenate([jnp.zeros((1, NRPF), jnp.float32), pos], axis=0)

    qs, k, v = _qkv(x2d, ln1_g, ln1_b, Wq, Wk, Wv)
    Rp = _relk(posp, Wrel)
    attn = _attn(qs, k, v, Rp, rcb.reshape(H, 1, DK), rpb.reshape(H, 1, DK))
    attn2 = attn.transpose(1, 0, 2).reshape(N, H * DV)
    x2, xn2, ti, gt = _post(x2d, attn2, Wo, bo, ln2_g, ln2_b, Wg)

    ef = ti.reshape(TK * N // 128, 128)
    p, te8 = _route(ef)
    pf = p.reshape(TK * N)
    p2 = pf.reshape(N, TK)
    p0 = p2[:, 0].reshape(1, N)
    p1 = p2[:, 1].reshape(1, N)
    te = te8[0, :NT]

    X_s = _sc_scatter(xn2, p0, p1)
    OUT_s = _gemm(X_s, We, be.reshape(NE, 1, D), te)
    OUT2 = _sc_gather(OUT_s, pf.reshape(1, TK * N))
    y = _combine(x2, OUT2.reshape(N, TK, D), gt)
    return y.reshape(1, N, D)
